# baseline (speedup 1.0000x reference)
import functools

import jax
import jax.numpy as jnp
from jax import lax
from jax.experimental import pallas as pl
from jax.experimental.pallas import tpu as pltpu
from jax.experimental.pallas import tpu_sc as plsc

BATCH = 16384
DIM = 64
NC = 2
NS = 16
NW = NC * NS
BPW = BATCH // NW
CHUNK = 128
NCHUNK = BPW // CHUNK


def _gather_body(u_hbm, i_hbm, n_hbm, ut_hbm, it_hbm,
                 u_out, i_out, n_out,
                 u_idx, i_idx, n_idx, u_rows, i_rows, n_rows, sem):
    wid = lax.axis_index("s") * NC + lax.axis_index("c")
    base = wid * BPW

    pltpu.sync_copy(u_hbm.at[pl.ds(base, BPW)], u_idx)
    pltpu.sync_copy(i_hbm.at[pl.ds(base, BPW)], i_idx)
    pltpu.sync_copy(n_hbm.at[pl.ds(base, BPW)], n_idx)

    copies = []
    for table, idx, rows in ((ut_hbm, u_idx, u_rows),
                             (it_hbm, i_idx, i_rows),
                             (it_hbm, n_idx, n_rows)):
        for j in range(NCHUNK):
            copies.append(pltpu.async_copy(
                table.at[idx.at[pl.ds(j * CHUNK, CHUNK)]],
                rows.at[pl.ds(j * CHUNK, CHUNK)],
                sem))
    for c in copies:
        c.wait()

    pltpu.sync_copy(u_rows, u_out.at[pl.ds(base, BPW)])
    pltpu.sync_copy(i_rows, i_out.at[pl.ds(base, BPW)])
    pltpu.sync_copy(n_rows, n_out.at[pl.ds(base, BPW)])


@jax.jit
def kernel(u, i, neg_i, u_table, i_table):
    out = jax.ShapeDtypeStruct((BATCH, DIM), jnp.float32)
    run = pl.kernel(
        _gather_body,
        out_type=(out, out, out),
        mesh=plsc.VectorSubcoreMesh(
            core_axis_name="c", subcore_axis_name="s",
            num_cores=NC, num_subcores=NS),
        scratch_types=[
            pltpu.VMEM((BPW,), jnp.int32),
            pltpu.VMEM((BPW,), jnp.int32),
            pltpu.VMEM((BPW,), jnp.int32),
            pltpu.VMEM((BPW, DIM), jnp.float32),
            pltpu.VMEM((BPW, DIM), jnp.float32),
            pltpu.VMEM((BPW, DIM), jnp.float32),
            pltpu.SemaphoreType.DMA,
        ],
        compiler_params=pltpu.CompilerParams(use_tc_tiling_on_sc=False),
    )
    return run(u, i, neg_i, u_table, i_table)


# speedup vs baseline: 1.0004x; 1.0004x over previous
import functools

import jax
import jax.numpy as jnp
from jax import lax
from jax.experimental import pallas as pl
from jax.experimental.pallas import tpu as pltpu
from jax.experimental.pallas import tpu_sc as plsc

BATCH = 16384
DIM = 64
NC = 2
NS = 16
NW = NC * NS
BPW = BATCH // NW
CHUNK = 128
NCHUNK = BPW // CHUNK


def _gather_body(idx_hbm, table_hbm, out_hbm, idx_v, rows_v, sem):
    wid = lax.axis_index("s") * NC + lax.axis_index("c")
    base = wid * BPW
    pltpu.sync_copy(idx_hbm.at[pl.ds(base, BPW)], idx_v)
    copies = [
        pltpu.async_copy(
            table_hbm.at[idx_v.at[pl.ds(j * CHUNK, CHUNK)]],
            rows_v.at[pl.ds(j * CHUNK, CHUNK)],
            sem)
        for j in range(NCHUNK)
    ]
    for c in copies:
        c.wait()
    pltpu.sync_copy(rows_v, out_hbm.at[pl.ds(base, BPW)])


def _lookup(idx, table):
    run = pl.kernel(
        _gather_body,
        out_type=jax.ShapeDtypeStruct((BATCH, DIM), jnp.float32),
        mesh=plsc.VectorSubcoreMesh(
            core_axis_name="c", subcore_axis_name="s",
            num_cores=NC, num_subcores=NS),
        scratch_types=[
            pltpu.VMEM((BPW,), jnp.int32),
            pltpu.VMEM((BPW, DIM), jnp.float32),
            pltpu.SemaphoreType.DMA,
        ],
        compiler_params=pltpu.CompilerParams(use_tc_tiling_on_sc=False),
    )
    return run(idx, table)


@jax.jit
def kernel(u, i, neg_i, u_table, i_table):
    return _lookup(u, u_table), _lookup(i, i_table), _lookup(neg_i, i_table)
